# Initial kernel scaffold; baseline (speedup 1.0000x reference)
import jax
import jax.numpy as jnp
from jax.experimental import pallas as pl


def kernel(x, noise_vals, noise_idx):
    raise NotImplementedError("write your pallas kernel here")


# speedup vs baseline: 4.2361x; 4.2361x over previous
import functools

import jax
import jax.numpy as jnp
from jax import lax
from jax.experimental import pallas as pl
from jax.experimental.pallas import tpu as pltpu
from jax.experimental.pallas import tpu_sc as plsc

B = 16384
N = 4096
K = 409
L = 16
NG = (K + L - 1) // L
TAIL = K - (NG - 1) * L
NW = 32
RPW = B // NW
C = 8
NCHUNK = RPW // C

_mesh = plsc.VectorSubcoreMesh(core_axis_name="c", subcore_axis_name="s")


@functools.partial(
    pl.kernel,
    mesh=_mesh,
    out_type=jax.ShapeDtypeStruct((B * N,), jnp.float32),
    scratch_types=[
        pltpu.VMEM((C * N,), jnp.float32),
        pltpu.VMEM((C * K + 8,), jnp.int32),
        pltpu.VMEM((C * K + 8,), jnp.float32),
    ],
    compiler_params=pltpu.CompilerParams(needs_layout_passes=False),
)
def _sp_noise(x_hbm, lin_hbm, val_hbm, out_hbm, xbuf, ibuf, vbuf):
    wid = lax.axis_index("s") * 2 + lax.axis_index("c")
    row0 = wid * RPW
    lane = lax.iota(jnp.int32, L)
    lane_lt_tail1 = lane < TAIL - 1
    lane_eq_tail1 = lane == TAIL - 1

    def chunk_body(g, carry):
        base = row0 + g * C
        pltpu.sync_copy(x_hbm.at[pl.ds(base * N, C * N)], xbuf)
        pltpu.sync_copy(lin_hbm.at[pl.ds(base * K, C * K)], ibuf.at[pl.ds(0, C * K)])
        pltpu.sync_copy(val_hbm.at[pl.ds(base * K, C * K)], vbuf.at[pl.ds(0, C * K)])
        for r in range(C):
            p0 = r * K
            for j in range(NG):
                lin = ibuf[pl.ds(p0 + j * L, L)]
                nxt = ibuf[pl.ds(p0 + j * L + 1, L)]
                vals = vbuf[pl.ds(p0 + j * L, L)]
                sp = jnp.where(vals < 0.5, jnp.float32(0.0), jnp.float32(1.0))
                neq = lin != nxt
                if j == NG - 1:
                    m = lane_eq_tail1 | (lane_lt_tail1 & neq)
                else:
                    m = neq
                tgt = (lin & (N - 1)) + jnp.full((L,), r * N, dtype=jnp.int32)
                plsc.store_scatter(xbuf, [tgt], sp, mask=m)
        pltpu.sync_copy(xbuf, out_hbm.at[pl.ds(base * N, C * N)])
        return carry

    lax.fori_loop(0, NCHUNK, chunk_body, 0)


def kernel(x, noise_vals, noise_idx):
    lin = (jnp.arange(B, dtype=jnp.int32)[:, None] * N + noise_idx).reshape(-1)
    s_lin, s_val = lax.sort((lin, noise_vals.reshape(-1)), dimension=0,
                            num_keys=1, is_stable=False)
    out = _sp_noise(x.reshape(-1), s_lin, s_val)
    return out.reshape(B, N)


# speedup vs baseline: 4.4274x; 1.0452x over previous
import functools

import jax
import jax.numpy as jnp
from jax import lax
from jax.experimental import pallas as pl
from jax.experimental.pallas import tpu as pltpu
from jax.experimental.pallas import tpu_sc as plsc

B = 16384
N = 4096
K = 409
L = 16
NG = (K + L - 1) // L
TAIL = K - (NG - 1) * L
NW = 32
RPW = B // NW
C = 8
NCHUNK = RPW // C

_mesh = plsc.VectorSubcoreMesh(core_axis_name="c", subcore_axis_name="s")


@functools.partial(
    pl.kernel,
    mesh=_mesh,
    out_type=jax.ShapeDtypeStruct((B * N,), jnp.float32),
    scratch_types=[
        pltpu.VMEM((C * N,), jnp.float32),
        pltpu.VMEM((C * K + 8,), jnp.int32),
        pltpu.VMEM((C * K + 8,), jnp.float32),
    ],
    compiler_params=pltpu.CompilerParams(needs_layout_passes=False),
)
def _sp_noise(x_hbm, lin_hbm, val_hbm, out_hbm, xbuf, ibuf, vbuf):
    wid = lax.axis_index("s") * 2 + lax.axis_index("c")
    row0 = wid * RPW
    lane = lax.iota(jnp.int32, L)
    lane_lt_tail1 = lane < TAIL - 1
    lane_eq_tail1 = lane == TAIL - 1

    def chunk_body(g, carry):
        base = row0 + g * C
        pltpu.sync_copy(x_hbm.at[pl.ds(base * N, C * N)], xbuf)
        pltpu.sync_copy(lin_hbm.at[pl.ds(base * K, C * K)], ibuf.at[pl.ds(0, C * K)])
        pltpu.sync_copy(val_hbm.at[pl.ds(base * K, C * K)], vbuf.at[pl.ds(0, C * K)])

        def row_body(r, rcarry):
            p0 = r * K
            for j in range(NG):
                lin = ibuf[pl.ds(p0 + j * L, L)]
                nxt = ibuf[pl.ds(p0 + j * L + 1, L)]
                vals = vbuf[pl.ds(p0 + j * L, L)]
                sp = jnp.where(vals < 0.5, jnp.float32(0.0), jnp.float32(1.0))
                neq = lin != nxt
                if j == NG - 1:
                    m = lane_eq_tail1 | (lane_lt_tail1 & neq)
                else:
                    m = neq
                plsc.store_scatter(xbuf, [lin & (C * N - 1)], sp, mask=m)
            return rcarry

        lax.fori_loop(0, C, row_body, 0)
        pltpu.sync_copy(xbuf, out_hbm.at[pl.ds(base * N, C * N)])
        return carry

    lax.fori_loop(0, NCHUNK, chunk_body, 0)


def kernel(x, noise_vals, noise_idx):
    lin = (jnp.arange(B, dtype=jnp.int32)[:, None] * N + noise_idx).reshape(-1)
    s_lin, s_val = lax.sort((lin, noise_vals.reshape(-1)), dimension=0,
                            num_keys=1, is_stable=False)
    s_tld = ((s_lin >> 15) << 15) | ((s_lin & 0x0F80) << 3) \
            | ((s_lin & 0x7000) >> 5) | (s_lin & 0x7F)
    x_t = x.reshape(B // 8, 8, N // 128, 128).transpose(0, 2, 1, 3).reshape(-1)
    out = _sp_noise(x_t, s_tld, s_val)
    out = out.reshape(B // 8, N // 128, 8, 128).transpose(0, 2, 1, 3)
    return out.reshape(B, N)


# speedup vs baseline: 4.5612x; 1.0302x over previous
import functools

import jax
import jax.numpy as jnp
from jax import lax
from jax.experimental import pallas as pl
from jax.experimental.pallas import tpu as pltpu
from jax.experimental.pallas import tpu_sc as plsc

B = 16384
N = 4096
K = 409
L = 16
NG = (K + L - 1) // L
TAIL = K - (NG - 1) * L
NW = 32
RPW = B // NW
C = 8
NCHUNK = RPW // C
NBUF = 3

_mesh = plsc.VectorSubcoreMesh(core_axis_name="c", subcore_axis_name="s")

_scratch = []
for _ in range(NBUF):
    _scratch += [
        pltpu.VMEM((C * N,), jnp.float32),
        pltpu.VMEM((C * K + 8,), jnp.int32),
        pltpu.VMEM((C * K + 8,), jnp.float32),
        pltpu.SemaphoreType.DMA,
        pltpu.SemaphoreType.DMA,
        pltpu.SemaphoreType.DMA,
        pltpu.SemaphoreType.DMA,
    ]


@functools.partial(
    pl.kernel,
    mesh=_mesh,
    out_type=jax.ShapeDtypeStruct((B * N,), jnp.float32),
    scratch_types=_scratch,
    compiler_params=pltpu.CompilerParams(needs_layout_passes=False),
)
def _sp_noise(x_hbm, lin_hbm, val_hbm, out_hbm, *bufs):
    sets = [tuple(bufs[7 * i:7 * i + 7]) for i in range(NBUF)]
    wid = lax.axis_index("s") * 2 + lax.axis_index("c")
    row0 = wid * RPW
    lane = lax.iota(jnp.int32, L)
    lane_lt_tail1 = lane < TAIL - 1
    lane_eq_tail1 = lane == TAIL - 1

    def in_copies(c, S):
        xb, ib, vb, sx, si, sv, so = S
        base = row0 + c * C
        return (
            pltpu.make_async_copy(x_hbm.at[pl.ds(base * N, C * N)], xb, sx),
            pltpu.make_async_copy(
                lin_hbm.at[pl.ds(base * K, C * K)], ib.at[pl.ds(0, C * K)], si),
            pltpu.make_async_copy(
                val_hbm.at[pl.ds(base * K, C * K)], vb.at[pl.ds(0, C * K)], sv),
        )

    def out_copy(c, S):
        xb, ib, vb, sx, si, sv, so = S
        base = row0 + c * C
        return pltpu.make_async_copy(xb, out_hbm.at[pl.ds(base * N, C * N)], so)

    def issue_in(c, S):
        for cp in in_copies(c, S):
            cp.start()

    def wait_in(c, S):
        for cp in in_copies(c, S):
            cp.wait()

    def compute(S):
        xb, ib, vb, sx, si, sv, so = S

        def row_body(r, rcarry):
            p0 = r * K
            for j in range(NG):
                lin = ib[pl.ds(p0 + j * L, L)]
                nxt = ib[pl.ds(p0 + j * L + 1, L)]
                vals = vb[pl.ds(p0 + j * L, L)]
                sp = jnp.where(vals < 0.5, jnp.float32(0.0), jnp.float32(1.0))
                neq = lin != nxt
                if j == NG - 1:
                    m = lane_eq_tail1 | (lane_lt_tail1 & neq)
                else:
                    m = neq
                plsc.store_scatter(xb, [lin & (C * N - 1)], sp, mask=m)
            return rcarry

        lax.fori_loop(0, C, row_body, 0)

    issue_in(0, sets[0])
    issue_in(1, sets[1])

    def loop_body(k, carry):
        for j in range(NBUF):
            c = NBUF * k + j
            S = sets[j]
            wait_in(c, S)
            compute(S)
            out_copy(c, S).start()
            Sn = sets[(j + 2) % NBUF]

            @pl.when(c >= 1)
            def _():
                out_copy(c - 1, Sn).wait()

            @pl.when(c <= NCHUNK - 3)
            def _():
                issue_in(c + 2, Sn)
        return carry

    lax.fori_loop(0, (NCHUNK - 1) // NBUF, loop_body, 0)

    c_last = NCHUNK - 1
    S = sets[c_last % NBUF]
    wait_in(c_last, S)
    compute(S)
    out_copy(c_last, S).start()
    out_copy(c_last - 1, sets[(c_last - 1) % NBUF]).wait()
    out_copy(c_last, S).wait()


def kernel(x, noise_vals, noise_idx):
    lin = (jnp.arange(B, dtype=jnp.int32)[:, None] * N + noise_idx).reshape(-1)
    s_lin, s_val = lax.sort((lin, noise_vals.reshape(-1)), dimension=0,
                            num_keys=1, is_stable=False)
    s_tld = ((s_lin >> 15) << 15) | ((s_lin & 0x0F80) << 3) \
            | ((s_lin & 0x7000) >> 5) | (s_lin & 0x7F)
    x_t = x.reshape(B // 8, 8, N // 128, 128).transpose(0, 2, 1, 3).reshape(-1)
    out = _sp_noise(x_t, s_tld, s_val)
    out = out.reshape(B // 8, N // 128, 8, 128).transpose(0, 2, 1, 3)
    return out.reshape(B, N)


# speedup vs baseline: 4.5989x; 1.0083x over previous
import functools

import jax
import jax.numpy as jnp
from jax import lax
from jax.experimental import pallas as pl
from jax.experimental.pallas import tpu as pltpu
from jax.experimental.pallas import tpu_sc as plsc

B = 16384
N = 4096
K = 409
L = 16
NG = (K + L - 1) // L
TAIL = K - (NG - 1) * L
NW = 32
RPW = B // NW
C = 8
NCHUNK = RPW // C
NBUF = 3

_mesh = plsc.VectorSubcoreMesh(core_axis_name="c", subcore_axis_name="s")

_scratch = []
for _ in range(NBUF):
    _scratch += [
        pltpu.VMEM((C * N,), jnp.float32),
        pltpu.VMEM((C * K + 8,), jnp.int32),
        pltpu.VMEM((C * K + 8,), jnp.float32),
        pltpu.SemaphoreType.DMA,
        pltpu.SemaphoreType.DMA,
        pltpu.SemaphoreType.DMA,
        pltpu.SemaphoreType.DMA,
    ]


@functools.partial(
    pl.kernel,
    mesh=_mesh,
    out_type=jax.ShapeDtypeStruct((B * N,), jnp.float32),
    scratch_types=_scratch,
    compiler_params=pltpu.CompilerParams(needs_layout_passes=False),
)
def _sp_noise(x_hbm, lin_hbm, val_hbm, out_hbm, *bufs):
    sets = [tuple(bufs[7 * i:7 * i + 7]) for i in range(NBUF)]
    wid = lax.axis_index("s") * 2 + lax.axis_index("c")
    row0 = wid * RPW
    lane = lax.iota(jnp.int32, L)
    lane_lt_tail1 = lane < TAIL - 1
    lane_eq_tail1 = lane == TAIL - 1

    def in_copies(c, S):
        xb, ib, vb, sx, si, sv, so = S
        base = row0 + c * C
        return (
            pltpu.make_async_copy(x_hbm.at[pl.ds(base * N, C * N)], xb, sx),
            pltpu.make_async_copy(
                lin_hbm.at[pl.ds(base * K, C * K)], ib.at[pl.ds(0, C * K)], si),
            pltpu.make_async_copy(
                val_hbm.at[pl.ds(base * K, C * K)], vb.at[pl.ds(0, C * K)], sv),
        )

    def out_copy(c, S):
        xb, ib, vb, sx, si, sv, so = S
        base = row0 + c * C
        return pltpu.make_async_copy(xb, out_hbm.at[pl.ds(base * N, C * N)], so)

    def issue_in(c, S):
        for cp in in_copies(c, S):
            cp.start()

    def wait_in(c, S):
        for cp in in_copies(c, S):
            cp.wait()

    def compute(S):
        xb, ib, vb, sx, si, sv, so = S

        def row_body(r, rcarry):
            p0 = r * K
            for j in range(NG):
                lin = ib[pl.ds(p0 + j * L, L)]
                nxt = ib[pl.ds(p0 + j * L + 1, L)]
                sp = vb[pl.ds(p0 + j * L, L)]
                neq = lin != nxt
                if j == NG - 1:
                    m = lane_eq_tail1 | (lane_lt_tail1 & neq)
                else:
                    m = neq
                plsc.store_scatter(xb, [lin & (C * N - 1)], sp, mask=m)
            return rcarry

        lax.fori_loop(0, C, row_body, 0)

    issue_in(0, sets[0])
    issue_in(1, sets[1])

    def loop_body(k, carry):
        for j in range(NBUF):
            c = NBUF * k + j
            S = sets[j]
            wait_in(c, S)
            compute(S)
            out_copy(c, S).start()
            Sn = sets[(j + 2) % NBUF]

            @pl.when(c >= 1)
            def _():
                out_copy(c - 1, Sn).wait()

            @pl.when(c <= NCHUNK - 3)
            def _():
                issue_in(c + 2, Sn)
        return carry

    lax.fori_loop(0, (NCHUNK - 1) // NBUF, loop_body, 0)

    c_last = NCHUNK - 1
    S = sets[c_last % NBUF]
    wait_in(c_last, S)
    compute(S)
    out_copy(c_last, S).start()
    out_copy(c_last - 1, sets[(c_last - 1) % NBUF]).wait()
    out_copy(c_last, S).wait()


def kernel(x, noise_vals, noise_idx):
    lin = (jnp.arange(B, dtype=jnp.int32)[:, None] * N + noise_idx).reshape(-1)
    spb = (noise_vals >= 0.5).astype(jnp.int16).reshape(-1)
    s_lin, s_spb = lax.sort((lin, spb), dimension=0,
                            num_keys=1, is_stable=False)
    s_val = s_spb.astype(jnp.float32)
    s_tld = ((s_lin >> 15) << 15) | ((s_lin & 0x0F80) << 3) \
            | ((s_lin & 0x7000) >> 5) | (s_lin & 0x7F)
    x_t = x.reshape(B // 8, 8, N // 128, 128).transpose(0, 2, 1, 3).reshape(-1)
    out = _sp_noise(x_t, s_tld, s_val)
    out = out.reshape(B // 8, N // 128, 8, 128).transpose(0, 2, 1, 3)
    return out.reshape(B, N)
